# baseline (speedup 1.0000x reference)
import functools

import jax
import jax.numpy as jnp
from jax import lax
from jax.experimental import pallas as pl
from jax.experimental.pallas import tpu as pltpu
from jax.experimental.pallas import tpu_sc as plsc

CH = 1664
SC_CPW = 15
TC_BLOCK = 4096
TAIL_BLOCK = 4096


def _tc_copy_body(src, dst):
    dst[...] = src[...]


def _tc_copy(x):
    d, n = x.shape
    spec = pl.BlockSpec((d, TC_BLOCK), lambda g: (0, g))
    return pl.pallas_call(
        _tc_copy_body,
        grid=(pl.cdiv(n, TC_BLOCK),),
        out_shape=jax.ShapeDtypeStruct(x.shape, x.dtype),
        in_specs=[spec],
        out_specs=spec,
    )(x)


def _sc_head_copy(x, head_cols, nw, nc, cpw):
    d, n = x.shape

    mesh = plsc.VectorSubcoreMesh(core_axis_name="c", subcore_axis_name="s")

    @functools.partial(
        pl.kernel,
        mesh=mesh,
        out_type=jax.ShapeDtypeStruct(x.shape, x.dtype),
        scratch_types=[
            pltpu.VMEM((d, CH), jnp.float32),
            pltpu.VMEM((d, CH), jnp.float32),
            pltpu.SemaphoreType.DMA((2,)),
            pltpu.SemaphoreType.DMA((2,)),
        ],
    )
    def sc_copy(src, dst, buf0, buf1, rsem, wsem):
        wid = lax.axis_index("s") * nc + lax.axis_index("c")
        bufs = (buf0, buf1)

        def rd(k):
            off = (wid * cpw + k) * CH
            return pltpu.make_async_copy(
                src.at[:, pl.ds(off, CH)], bufs[k % 2], rsem.at[k % 2]
            )

        def wr(k):
            off = (wid * cpw + k) * CH
            return pltpu.make_async_copy(
                bufs[k % 2], dst.at[:, pl.ds(off, CH)], wsem.at[k % 2]
            )

        rd(0).start()
        for k in range(cpw):
            if k + 1 < cpw:
                if k >= 1:
                    wr(k - 1).wait()
                rd(k + 1).start()
            rd(k).wait()
            wr(k).start()
        if cpw >= 2:
            wr(cpw - 2).wait()
        wr(cpw - 1).wait()

    return sc_copy(x)


def _tc_tail_fill(partial, src, head_cols):
    d, n = src.shape
    first_blk = head_cols // TAIL_BLOCK
    nblk = pl.cdiv(n - head_cols, TAIL_BLOCK)
    spec = pl.BlockSpec((d, TAIL_BLOCK), lambda g: (0, first_blk + g))

    def body(_alias, src_blk, out_blk):
        out_blk[...] = src_blk[...]

    return pl.pallas_call(
        body,
        grid=(nblk,),
        out_shape=jax.ShapeDtypeStruct(src.shape, src.dtype),
        in_specs=[pl.BlockSpec(memory_space=pl.ANY), spec],
        out_specs=spec,
        input_output_aliases={0: 0},
    )(partial, src)


def kernel(user_emb, item_emb):
    ut = user_emb.T
    it = item_emb.T
    d, n = ut.shape

    info = plsc.get_sparse_core_info()
    nw = info.num_cores * info.num_subcores
    nc = info.num_cores
    head_cols = SC_CPW * nw * CH
    assert head_cols % TAIL_BLOCK == 0 and head_cols < n

    out_ut = _tc_copy(ut)
    it_head = _sc_head_copy(it, head_cols, nw, nc, SC_CPW)
    out_it = _tc_tail_fill(it_head, it, head_cols)
    return out_ut.T, out_it.T


# speedup vs baseline: 1.7386x; 1.7386x over previous
import jax
import jax.numpy as jnp
from jax.experimental import pallas as pl
from jax.experimental.pallas import tpu as pltpu

BLOCK_COLS = 32768


def _copy_body(u_in, i_in, u_out, i_out):
    u_out[...] = u_in[...]
    i_out[...] = i_in[...]


def kernel(user_emb, item_emb):
    ut = user_emb.T
    it = item_emb.T
    d, n = ut.shape
    grid = (pl.cdiv(n, BLOCK_COLS),)
    spec = pl.BlockSpec((d, BLOCK_COLS), lambda g: (0, g))
    out_ut, out_it = pl.pallas_call(
        _copy_body,
        grid=grid,
        out_shape=(
            jax.ShapeDtypeStruct(ut.shape, ut.dtype),
            jax.ShapeDtypeStruct(it.shape, it.dtype),
        ),
        in_specs=[spec, spec],
        out_specs=[spec, spec],
    )(ut, it)
    return out_ut.T, out_it.T
